# baseline (device time: 37225 ns/iter reference)
import jax
import jax.numpy as jnp
from jax import lax
from jax.experimental import pallas as pl
from jax.experimental.pallas import tpu as pltpu

N_DEV = 8

_MASKS = [(1, 1, 1), (1, 1, 0), (1, 0, 1), (0, 1, 1), (1, 0, 0), (0, 1, 0),
          (0, 0, 1)]


def _xyz_to_pos(x, y, z):
    return 4 * z + 2 * y + (x ^ y)


def kernel(x, w_mat, scale_x, scale_w):
    m_per, k = x.shape
    _, n = w_mat.shape
    n_per = n // N_DEV

    def body(peers_ref, x_ref, w_ref, sx_ref, sw_ref, out_ref,
             send_buf, recv_buf, send_sc, recv_sc,
             send_sems, recv_sems, sc_send_sems, sc_recv_sems):
        my = peers_ref[0]
        scale = sx_ref[0] * sw_ref[0]

        barrier_sem = pltpu.get_barrier_semaphore()
        for d in range(1, N_DEV):
            pl.semaphore_signal(barrier_sem, inc=1, device_id=(peers_ref[d],),
                                device_id_type=pl.DeviceIdType.MESH)
        pl.semaphore_wait(barrier_sem, N_DEV - 1)

        sends = []
        for d in range(1, N_DEV):
            peer = peers_ref[d]
            wj = w_ref[:, pl.ds(peer * n_per, n_per)]
            acc = jnp.dot(x_ref[:, :], wj, preferred_element_type=jnp.int32)
            y = jnp.maximum(acc.astype(jnp.float32), 0.0)
            rowmax = jnp.max(y, axis=1, keepdims=True)
            q = jnp.round(y * (127.0 / jnp.maximum(rowmax, 1.0)))
            send_buf[d, :, :] = q.astype(jnp.int8)
            send_sc[d, :, :] = rowmax * ((1.0 / 127.0) * scale)
            for (src, dst, ssem, rsem) in (
                (send_buf, recv_buf, send_sems, recv_sems),
                (send_sc, recv_sc, sc_send_sems, sc_recv_sems),
            ):
                rdma = pltpu.make_async_remote_copy(
                    src_ref=src.at[d],
                    dst_ref=dst.at[d],
                    send_sem=ssem.at[d],
                    recv_sem=rsem.at[d],
                    device_id=(peer,),
                    device_id_type=pl.DeviceIdType.MESH,
                )
                rdma.start()
                sends.append(rdma)

        wj = w_ref[:, pl.ds(my * n_per, n_per)]
        acc = jnp.dot(x_ref[:, :], wj, preferred_element_type=jnp.int32)
        out_ref[pl.ds(my * m_per, m_per), :] = jnp.maximum(
            acc.astype(jnp.float32) * scale, 0.0
        )

        for d in range(1, N_DEV):
            peer = peers_ref[d]
            for (src, dst, ssem, rsem) in (
                (send_buf, recv_buf, send_sems, recv_sems),
                (send_sc, recv_sc, sc_send_sems, sc_recv_sems),
            ):
                recv = pltpu.make_async_remote_copy(
                    src_ref=src.at[d],
                    dst_ref=dst.at[d],
                    send_sem=ssem.at[d],
                    recv_sem=rsem.at[d],
                    device_id=(my,),
                    device_id_type=pl.DeviceIdType.MESH,
                )
                recv.wait_recv()
            out_ref[pl.ds(peer * m_per, m_per), :] = (
                recv_buf[d].astype(jnp.float32) * recv_sc[d]
            )
        for rdma in sends:
            rdma.wait_send()

    my = lax.axis_index("i")
    mz, g = my // 4, my % 4
    my_y = g // 2
    mx = (g % 2) ^ my_y
    peers = [my]
    for tx, ty, tz in _MASKS:
        peers.append(_xyz_to_pos(mx ^ tx, my_y ^ ty, mz ^ tz))
    peers = jnp.stack([p.astype(jnp.int32) for p in peers])

    out_shape = jax.ShapeDtypeStruct((m_per * N_DEV, n_per), jnp.float32)
    return pl.pallas_call(
        body,
        out_shape=out_shape,
        in_specs=[
            pl.BlockSpec(memory_space=pltpu.SMEM),
            pl.BlockSpec(memory_space=pltpu.VMEM),
            pl.BlockSpec(memory_space=pltpu.VMEM),
            pl.BlockSpec(memory_space=pltpu.SMEM),
            pl.BlockSpec(memory_space=pltpu.SMEM),
        ],
        out_specs=pl.BlockSpec(memory_space=pltpu.VMEM),
        scratch_shapes=[
            pltpu.VMEM((N_DEV, m_per, n_per), jnp.int8),
            pltpu.VMEM((N_DEV, m_per, n_per), jnp.int8),
            pltpu.VMEM((N_DEV, m_per, 1), jnp.float32),
            pltpu.VMEM((N_DEV, m_per, 1), jnp.float32),
            pltpu.SemaphoreType.DMA((N_DEV,)),
            pltpu.SemaphoreType.DMA((N_DEV,)),
            pltpu.SemaphoreType.DMA((N_DEV,)),
            pltpu.SemaphoreType.DMA((N_DEV,)),
        ],
        compiler_params=pltpu.CompilerParams(collective_id=0),
    )(peers, x, w_mat, scale_x, scale_w)


# device time: 34565 ns/iter; 1.0770x vs baseline; 1.0770x over previous
import jax
import jax.numpy as jnp
from jax import lax
from jax.experimental import pallas as pl
from jax.experimental.pallas import tpu as pltpu

N_DEV = 8

_MASKS = [(1, 1, 1), (1, 1, 0), (1, 0, 1), (0, 1, 1), (1, 0, 0), (0, 1, 0),
          (0, 0, 1)]


def _xyz_to_pos(x, y, z):
    return 4 * z + 2 * y + (x ^ y)


def kernel(x, w_mat, scale_x, scale_w):
    m_per, k = x.shape
    _, n = w_mat.shape
    n_per = n // N_DEV
    def body(peers_ref, x_ref, w_ref, sx_ref, sw_ref, out_ref,
             send_buf, recv_buf, send_sc, recv_sc,
             send_sems, recv_sems, sc_send_sems, sc_recv_sems):
        my = peers_ref[0]
        scale = sx_ref[0] * sw_ref[0]

        barrier_sem = pltpu.get_barrier_semaphore()
        for d in range(1, N_DEV):
            pl.semaphore_signal(barrier_sem, inc=1, device_id=(peers_ref[d],),
                                device_id_type=pl.DeviceIdType.MESH)
        pl.semaphore_wait(barrier_sem, N_DEV - 1)

        sends = []
        for d in range(1, N_DEV):
            peer = peers_ref[d]
            wj = w_ref[:, pl.ds(peer * n_per, n_per)]
            acc = jnp.dot(x_ref[:, :], wj, preferred_element_type=jnp.int32)
            y = jnp.maximum(acc.astype(jnp.float32), 0.0)
            rowmax = jnp.max(y, axis=1)
            q = jnp.round(y * (127.0 / jnp.maximum(rowmax, 1.0))[:, None])
            send_buf[d, :, :] = q.astype(jnp.int8)
            send_sc[d, 0, :] = rowmax * ((1.0 / 127.0) * scale)
            for (src, dst, ssem, rsem) in (
                (send_buf, recv_buf, send_sems, recv_sems),
                (send_sc, recv_sc, sc_send_sems, sc_recv_sems),
            ):
                rdma = pltpu.make_async_remote_copy(
                    src_ref=src.at[d],
                    dst_ref=dst.at[d],
                    send_sem=ssem.at[d],
                    recv_sem=rsem.at[d],
                    device_id=(peer,),
                    device_id_type=pl.DeviceIdType.MESH,
                )
                rdma.start()
                sends.append(rdma)

        wj = w_ref[:, pl.ds(my * n_per, n_per)]
        acc = jnp.dot(x_ref[:, :], wj, preferred_element_type=jnp.int32)
        out_ref[pl.ds(my * m_per, m_per), :] = jnp.maximum(
            acc.astype(jnp.float32) * scale, 0.0
        )

        for d in range(1, N_DEV):
            peer = peers_ref[d]
            for (src, dst, ssem, rsem) in (
                (send_buf, recv_buf, send_sems, recv_sems),
                (send_sc, recv_sc, sc_send_sems, sc_recv_sems),
            ):
                recv = pltpu.make_async_remote_copy(
                    src_ref=src.at[d],
                    dst_ref=dst.at[d],
                    send_sem=ssem.at[d],
                    recv_sem=rsem.at[d],
                    device_id=(my,),
                    device_id_type=pl.DeviceIdType.MESH,
                )
                recv.wait_recv()
            out_ref[pl.ds(peer * m_per, m_per), :] = (
                recv_buf[d].astype(jnp.float32) * recv_sc[d, 0, :][:, None]
            )
        for rdma in sends:
            rdma.wait_send()

    my = lax.axis_index("i")
    mz, g = my // 4, my % 4
    my_y = g // 2
    mx = (g % 2) ^ my_y
    peers = [my]
    for tx, ty, tz in _MASKS:
        peers.append(_xyz_to_pos(mx ^ tx, my_y ^ ty, mz ^ tz))
    peers = jnp.stack([p.astype(jnp.int32) for p in peers])

    out_shape = jax.ShapeDtypeStruct((m_per * N_DEV, n_per), jnp.float32)
    return pl.pallas_call(
        body,
        out_shape=out_shape,
        in_specs=[
            pl.BlockSpec(memory_space=pltpu.SMEM),
            pl.BlockSpec(memory_space=pltpu.VMEM),
            pl.BlockSpec(memory_space=pltpu.VMEM),
            pl.BlockSpec(memory_space=pltpu.SMEM),
            pl.BlockSpec(memory_space=pltpu.SMEM),
        ],
        out_specs=pl.BlockSpec(memory_space=pltpu.VMEM),
        scratch_shapes=[
            pltpu.VMEM((N_DEV, m_per, n_per), jnp.int8),
            pltpu.VMEM((N_DEV, m_per, n_per), jnp.int8),
            pltpu.VMEM((N_DEV, 1, m_per), jnp.float32),
            pltpu.VMEM((N_DEV, 1, m_per), jnp.float32),
            pltpu.SemaphoreType.DMA((N_DEV,)),
            pltpu.SemaphoreType.DMA((N_DEV,)),
            pltpu.SemaphoreType.DMA((N_DEV,)),
            pltpu.SemaphoreType.DMA((N_DEV,)),
        ],
        compiler_params=pltpu.CompilerParams(collective_id=0),
    )(peers, x, w_mat, scale_x, scale_w)


# device time: 34220 ns/iter; 1.0878x vs baseline; 1.0101x over previous
import jax
import jax.numpy as jnp
from jax import lax
from jax.experimental import pallas as pl
from jax.experimental.pallas import tpu as pltpu

N_DEV = 8

_MASKS = [(1, 1, 1), (1, 1, 0), (1, 0, 1), (0, 1, 1), (1, 0, 0), (0, 1, 0),
          (0, 0, 1)]


def _xyz_to_pos(x, y, z):
    return 4 * z + 2 * y + (x ^ y)


def kernel(x, w_mat, scale_x, scale_w):
    m_per, k = x.shape
    _, n = w_mat.shape
    n_per = n // N_DEV
    sc_rows = (2 * m_per) // n_per
    m_pack = m_per + sc_rows

    def body(peers_ref, x_ref, w_ref, sx_ref, sw_ref, out_ref,
             send_buf, recv_buf, send_sems, recv_sems):
        my = peers_ref[0]
        scale = sx_ref[0] * sw_ref[0]

        barrier_sem = pltpu.get_barrier_semaphore()
        for d in range(1, N_DEV):
            pl.semaphore_signal(barrier_sem, inc=1, device_id=(peers_ref[d],),
                                device_id_type=pl.DeviceIdType.MESH)
        pl.semaphore_wait(barrier_sem, N_DEV - 1)

        sends = []
        for d in range(1, N_DEV):
            peer = peers_ref[d]
            wj = w_ref[:, pl.ds(peer * n_per, n_per)]
            acc = jnp.dot(x_ref[:, :], wj, preferred_element_type=jnp.int32)
            y = jnp.maximum(acc.astype(jnp.float32), 0.0)
            rowmax = jnp.max(y, axis=1)
            q = jnp.round(y * (127.0 / jnp.maximum(rowmax, 1.0))[:, None])
            send_buf[d, :m_per, :] = q.astype(jnp.int8)
            rm = jnp.maximum(rowmax * ((1.0 / 127.0) * scale), 1e-30)
            e = jnp.ceil(jnp.log2(rm))
            mant = jnp.round(rm * jnp.exp2(-e) * 127.0)
            send_buf[d, m_per:m_per + 2, :] = (
                e.astype(jnp.int8).reshape(2, n_per))
            send_buf[d, m_per + 2:, :] = (
                mant.astype(jnp.int8).reshape(2, n_per))
            rdma = pltpu.make_async_remote_copy(
                src_ref=send_buf.at[d],
                dst_ref=recv_buf.at[d],
                send_sem=send_sems.at[d],
                recv_sem=recv_sems.at[d],
                device_id=(peer,),
                device_id_type=pl.DeviceIdType.MESH,
            )
            rdma.start()
            sends.append(rdma)

        wj = w_ref[:, pl.ds(my * n_per, n_per)]
        acc = jnp.dot(x_ref[:, :], wj, preferred_element_type=jnp.int32)
        out_ref[pl.ds(my * m_per, m_per), :] = jnp.maximum(
            acc.astype(jnp.float32) * scale, 0.0
        )

        for d in range(1, N_DEV):
            peer = peers_ref[d]
            recv = pltpu.make_async_remote_copy(
                src_ref=send_buf.at[d],
                dst_ref=recv_buf.at[d],
                send_sem=send_sems.at[d],
                recv_sem=recv_sems.at[d],
                device_id=(my,),
                device_id_type=pl.DeviceIdType.MESH,
            )
            recv.wait_recv()
            e = recv_buf[d, m_per:m_per + 2, :].reshape(m_per).astype(jnp.float32)
            mant = recv_buf[d, m_per + 2:, :].reshape(m_per).astype(jnp.float32)
            rm = mant * jnp.exp2(e) * (1.0 / 127.0)
            out_ref[pl.ds(peer * m_per, m_per), :] = (
                recv_buf[d, :m_per, :].astype(jnp.float32) * rm[:, None]
            )
        for rdma in sends:
            rdma.wait_send()

    my = lax.axis_index("i")
    mz, g = my // 4, my % 4
    my_y = g // 2
    mx = (g % 2) ^ my_y
    peers = [my]
    for tx, ty, tz in _MASKS:
        peers.append(_xyz_to_pos(mx ^ tx, my_y ^ ty, mz ^ tz))
    peers = jnp.stack([p.astype(jnp.int32) for p in peers])

    out_shape = jax.ShapeDtypeStruct((m_per * N_DEV, n_per), jnp.float32)
    return pl.pallas_call(
        body,
        out_shape=out_shape,
        in_specs=[
            pl.BlockSpec(memory_space=pltpu.SMEM),
            pl.BlockSpec(memory_space=pltpu.VMEM),
            pl.BlockSpec(memory_space=pltpu.VMEM),
            pl.BlockSpec(memory_space=pltpu.SMEM),
            pl.BlockSpec(memory_space=pltpu.SMEM),
        ],
        out_specs=pl.BlockSpec(memory_space=pltpu.VMEM),
        scratch_shapes=[
            pltpu.VMEM((N_DEV, m_pack, n_per), jnp.int8),
            pltpu.VMEM((N_DEV, m_pack, n_per), jnp.int8),
            pltpu.SemaphoreType.DMA((N_DEV,)),
            pltpu.SemaphoreType.DMA((N_DEV,)),
        ],
        compiler_params=pltpu.CompilerParams(collective_id=0),
    )(peers, x, w_mat, scale_x, scale_w)


# device time: 32872 ns/iter; 1.1324x vs baseline; 1.0410x over previous
import jax
import jax.numpy as jnp
from jax import lax
from jax.experimental import pallas as pl
from jax.experimental.pallas import tpu as pltpu

N_DEV = 8

_MASKS = [(1, 1, 1), (1, 1, 0), (1, 0, 1), (0, 1, 1), (1, 0, 0), (0, 1, 0),
          (0, 0, 1)]


def _xyz_to_pos(x, y, z):
    return 4 * z + 2 * y + (x ^ y)


def kernel(x, w_mat, scale_x, scale_w):
    m_per, k = x.shape
    _, n = w_mat.shape
    n_per = n // N_DEV

    def body(peers_ref, x_ref, w_ref, sx_ref, sw_ref, out_ref,
             send_buf, recv_buf, send_sems, recv_sems):
        my = peers_ref[0]
        scale = sx_ref[0] * sw_ref[0]

        barrier_sem = pltpu.get_barrier_semaphore()
        for d in range(1, N_DEV):
            pl.semaphore_signal(barrier_sem, inc=1, device_id=(peers_ref[d],),
                                device_id_type=pl.DeviceIdType.MESH)
        pl.semaphore_wait(barrier_sem, N_DEV - 1)

        sends = []
        for d in range(1, N_DEV):
            peer = peers_ref[d]
            wj = w_ref[:, pl.ds(peer * n_per, n_per)]
            acc = jnp.dot(x_ref[:, :], wj, preferred_element_type=jnp.int32)
            y = jnp.maximum(acc.astype(jnp.float32) * scale, 0.0)
            send_buf[d, :, :] = y.astype(jnp.bfloat16)
            rdma = pltpu.make_async_remote_copy(
                src_ref=send_buf.at[d],
                dst_ref=recv_buf.at[d],
                send_sem=send_sems.at[d],
                recv_sem=recv_sems.at[d],
                device_id=(peer,),
                device_id_type=pl.DeviceIdType.MESH,
            )
            rdma.start()
            sends.append(rdma)

        wj = w_ref[:, pl.ds(my * n_per, n_per)]
        acc = jnp.dot(x_ref[:, :], wj, preferred_element_type=jnp.int32)
        out_ref[pl.ds(my * m_per, m_per), :] = jnp.maximum(
            acc.astype(jnp.float32) * scale, 0.0
        )

        for d in range(1, N_DEV):
            peer = peers_ref[d]
            recv = pltpu.make_async_remote_copy(
                src_ref=send_buf.at[d],
                dst_ref=recv_buf.at[d],
                send_sem=send_sems.at[d],
                recv_sem=recv_sems.at[d],
                device_id=(my,),
                device_id_type=pl.DeviceIdType.MESH,
            )
            recv.wait_recv()
            out_ref[pl.ds(peer * m_per, m_per), :] = recv_buf[d].astype(jnp.float32)
        for rdma in sends:
            rdma.wait_send()

    my = lax.axis_index("i")
    mz, g = my // 4, my % 4
    my_y = g // 2
    mx = (g % 2) ^ my_y
    peers = [my]
    for tx, ty, tz in _MASKS:
        peers.append(_xyz_to_pos(mx ^ tx, my_y ^ ty, mz ^ tz))
    peers = jnp.stack([p.astype(jnp.int32) for p in peers])

    out_shape = jax.ShapeDtypeStruct((m_per * N_DEV, n_per), jnp.float32)
    return pl.pallas_call(
        body,
        out_shape=out_shape,
        in_specs=[
            pl.BlockSpec(memory_space=pltpu.SMEM),
            pl.BlockSpec(memory_space=pltpu.VMEM),
            pl.BlockSpec(memory_space=pltpu.VMEM),
            pl.BlockSpec(memory_space=pltpu.SMEM),
            pl.BlockSpec(memory_space=pltpu.SMEM),
        ],
        out_specs=pl.BlockSpec(memory_space=pltpu.VMEM),
        scratch_shapes=[
            pltpu.VMEM((N_DEV, m_per, n_per), jnp.bfloat16),
            pltpu.VMEM((N_DEV, m_per, n_per), jnp.bfloat16),
            pltpu.SemaphoreType.DMA((N_DEV,)),
            pltpu.SemaphoreType.DMA((N_DEV,)),
        ],
        compiler_params=pltpu.CompilerParams(collective_id=0),
    )(peers, x, w_mat, scale_x, scale_w)
